# baseline (speedup 1.0000x reference)
import functools

import jax
import jax.numpy as jnp
from jax import lax
from jax.experimental import pallas as pl
from jax.experimental.pallas import tpu as pltpu
from jax.experimental.pallas import tpu_sc as plsc

_info = plsc.get_sparse_core_info()
_NC, _NS = _info.num_cores, _info.num_subcores
_NW = _NC * _NS

_CHUNK_ROWS = 16
_NB = 4


@functools.partial(jax.jit, static_argnums=(0, 1))
def _copy_rows(seq_len, hidden, emb_table):
    rows_per_w = seq_len // _NW
    nch = rows_per_w // _CHUNK_ROWS
    mesh = plsc.VectorSubcoreMesh(core_axis_name="c", subcore_axis_name="s")

    @functools.partial(
        pl.kernel,
        mesh=mesh,
        out_type=jax.ShapeDtypeStruct((seq_len, hidden), jnp.float32),
        scratch_types=(
            [pltpu.VMEM((_CHUNK_ROWS, hidden), jnp.float32)] * 2
            + [pltpu.VMEM_SHARED((_NS, _CHUNK_ROWS, hidden), jnp.float32)] * 2
            + [pltpu.SemaphoreType.DMA] * (2 * _NB)
        ),
    )
    def k(table_hbm, out_hbm, tb0, tb1, sb0, sb1, *sems):
        sid = lax.axis_index("s")
        wid = sid * _NC + lax.axis_index("c")
        base = wid * rows_per_w
        bufs = (tb0, tb1, sb0.at[sid], sb1.at[sid])
        sin = sems[:_NB]
        sout = sems[_NB:]
        in_h = {}
        out_h = {}

        def start_in(c):
            b = c % _NB
            in_h[c] = pltpu.async_copy(
                table_hbm.at[pl.ds(base + c * _CHUNK_ROWS, _CHUNK_ROWS)],
                bufs[b],
                sin[b],
            )

        def start_out(c):
            b = c % _NB
            out_h[c] = pltpu.async_copy(
                bufs[b],
                out_hbm.at[pl.ds(base + c * _CHUNK_ROWS, _CHUNK_ROWS)],
                sout[b],
            )

        for c in range(min(_NB, nch)):
            start_in(c)
        for c in range(nch):
            in_h[c].wait()
            start_out(c)
            if c + _NB < nch:
                out_h[c].wait()
                start_in(c + _NB)
        for c in range(max(0, nch - _NB), nch):
            out_h[c].wait()

    return k(emb_table)


def kernel(x, emb_table):
    seq_len = x.shape[1]
    hidden = emb_table.shape[1]
    out = _copy_rows(seq_len, hidden, emb_table)
    return out[None]


# speedup vs baseline: 1.0204x; 1.0204x over previous
import functools

import jax
import jax.numpy as jnp
from jax import lax
from jax.experimental import pallas as pl
from jax.experimental.pallas import tpu as pltpu
from jax.experimental.pallas import tpu_sc as plsc

_info = plsc.get_sparse_core_info()
_NC, _NS = _info.num_cores, _info.num_subcores
_NW = _NC * _NS

_CHUNK_ROWS = 64


@functools.partial(jax.jit, static_argnums=(0, 1))
def _copy_rows(seq_len, hidden, emb_table):
    rows_per_w = seq_len // _NW
    nch = rows_per_w // _CHUNK_ROWS
    mesh = plsc.VectorSubcoreMesh(core_axis_name="c", subcore_axis_name="s")

    @functools.partial(
        pl.kernel,
        mesh=mesh,
        out_type=jax.ShapeDtypeStruct((seq_len, hidden), jnp.float32),
        scratch_types=[
            pltpu.VMEM((_CHUNK_ROWS, hidden), jnp.float32),
            pltpu.SemaphoreType.DMA,
            pltpu.SemaphoreType.DMA,
        ],
    )
    def k(table_hbm, out_hbm, tb, si, so):
        sid = lax.axis_index("s")
        wid = sid * _NC + lax.axis_index("c")
        base = wid * rows_per_w
        for c in range(nch):
            lo = base + c * _CHUNK_ROWS
            pltpu.async_copy(
                table_hbm.at[pl.ds(lo, _CHUNK_ROWS)], tb, si
            ).wait()
            pltpu.async_copy(
                tb, out_hbm.at[pl.ds(lo, _CHUNK_ROWS)], so
            ).wait()

    return k(emb_table)


def kernel(x, emb_table):
    seq_len = x.shape[1]
    hidden = emb_table.shape[1]
    out = _copy_rows(seq_len, hidden, emb_table)
    return out[None]


# speedup vs baseline: 1.0260x; 1.0055x over previous
import functools

import jax
import jax.numpy as jnp
from jax import lax
from jax.experimental import pallas as pl
from jax.experimental.pallas import tpu as pltpu
from jax.experimental.pallas import tpu_sc as plsc

_info = plsc.get_sparse_core_info()
_NC, _NS = _info.num_cores, _info.num_subcores
_NW = _NC * _NS

_CHUNK_ROWS = 32


@functools.partial(jax.jit, static_argnums=(0, 1))
def _copy_rows(seq_len, hidden, emb_table):
    rows_per_w = seq_len // _NW
    nch = rows_per_w // _CHUNK_ROWS
    mesh = plsc.VectorSubcoreMesh(core_axis_name="c", subcore_axis_name="s")

    @functools.partial(
        pl.kernel,
        mesh=mesh,
        out_type=jax.ShapeDtypeStruct((seq_len, hidden), jnp.float32),
        scratch_types=[
            pltpu.VMEM((_CHUNK_ROWS, hidden), jnp.float32),
            pltpu.VMEM_SHARED((_NS, _CHUNK_ROWS, hidden), jnp.float32),
            pltpu.SemaphoreType.DMA,
            pltpu.SemaphoreType.DMA,
            pltpu.SemaphoreType.DMA,
            pltpu.SemaphoreType.DMA,
        ],
    )
    def k(table_hbm, out_hbm, tb, sb, si0, si1, so0, so1):
        sid = lax.axis_index("s")
        wid = sid * _NC + lax.axis_index("c")
        base = wid * rows_per_w
        bufs = (tb, sb.at[sid])
        sin = (si0, si1)
        sout = (so0, so1)
        in_h = {}
        out_h = {}

        def start_in(c):
            b = c % 2
            in_h[c] = pltpu.async_copy(
                table_hbm.at[pl.ds(base + c * _CHUNK_ROWS, _CHUNK_ROWS)],
                bufs[b],
                sin[b],
            )

        def start_out(c):
            b = c % 2
            out_h[c] = pltpu.async_copy(
                bufs[b],
                out_hbm.at[pl.ds(base + c * _CHUNK_ROWS, _CHUNK_ROWS)],
                sout[b],
            )

        start_in(0)
        if nch > 1:
            start_in(1)
        for c in range(nch):
            in_h[c].wait()
            start_out(c)
            if c + 2 < nch:
                out_h[c].wait()
                start_in(c + 2)
        for c in range(max(0, nch - 2), nch):
            out_h[c].wait()

    return k(emb_table)


def kernel(x, emb_table):
    seq_len = x.shape[1]
    hidden = emb_table.shape[1]
    out = _copy_rows(seq_len, hidden, emb_table)
    return out[None]
